# Initial kernel scaffold; baseline (speedup 1.0000x reference)
import jax
import jax.numpy as jnp
from jax.experimental import pallas as pl


def kernel(x, edge_index, no_pedestrians, params):
    raise NotImplementedError("write your pallas kernel here")


# speedup vs baseline: 82.6241x; 82.6241x over previous
import functools

import jax
import jax.numpy as jnp
from jax import lax
from jax.experimental import pallas as pl
from jax.experimental.pallas import tpu as pltpu
from jax.experimental.pallas import tpu_sc as plsc

_N = 512
_F = 512
_E = 131072
_NC = 2
_NS = 16
_NW = _NC * _NS
_EPW = _E // _NW
_CHUNK = 128
_NROW = _EPW // _CHUNK
_SLICE = (_N * _N) // _NS


def _sc_count_body(src_hbm, dst_hbm, zeros_hbm, ones_hbm, out_hbm,
                   src_v, dst_v, idx_v, ones_v, shared):
    c = lax.axis_index("c")
    s = lax.axis_index("s")
    wid = s * _NC + c
    base = wid * _EPW
    pltpu.sync_copy(src_hbm.at[pl.ds(base, _EPW)], src_v)
    pltpu.sync_copy(dst_hbm.at[pl.ds(base, _EPW)], dst_v)
    pltpu.sync_copy(ones_hbm, ones_v)
    pltpu.sync_copy(zeros_hbm.at[pl.ds(s * _SLICE, _SLICE)],
                    shared.at[pl.ds(s * _SLICE, _SLICE)])

    def body(i, carry):
        sv = src_v[pl.ds(i * 16, 16)]
        dv = dst_v[pl.ds(i * 16, 16)]
        row = i // 8
        col = (i % 8) * 16
        idx_v[row, pl.ds(col, 16)] = dv * _N + sv
        return carry

    lax.fori_loop(0, _EPW // 16, body, 0)

    plsc.subcore_barrier()
    for j in range(_NROW):
        pltpu.sync_copy(ones_v, shared.at[idx_v.at[j]], add=True)
    plsc.subcore_barrier()
    pltpu.sync_copy(shared.at[pl.ds(s * _SLICE, _SLICE)],
                    out_hbm.at[c, pl.ds(s * _SLICE, _SLICE)])


@functools.lru_cache(maxsize=1)
def _make_sc_count():
    return functools.partial(
        pl.kernel,
        mesh=plsc.VectorSubcoreMesh(core_axis_name="c", subcore_axis_name="s"),
        out_type=jax.ShapeDtypeStruct((_NC, _N * _N), jnp.float32),
        scratch_types=[
            pltpu.VMEM((_EPW,), jnp.int32),
            pltpu.VMEM((_EPW,), jnp.int32),
            pltpu.VMEM((_NROW, _CHUNK), jnp.int32),
            pltpu.VMEM((_CHUNK,), jnp.float32),
            pltpu.VMEM_SHARED((_N * _N,), jnp.float32),
        ],
    )(_sc_count_body)


def _tc_body(counts_ref, x_ref, w_ref, b_ref, wj_ref, bj_ref, out_ref):
    count = counts_ref[0] + counts_ref[1]
    deg = jnp.sum(count, axis=1, keepdims=True) + 2.0
    dinv = lax.rsqrt(deg)
    di2 = 2.0 * dinv * dinv

    def mm(a, b):
        return jnp.dot(a, b, preferred_element_type=jnp.float32)

    def gcn(z, k):
        zw = mm(z, w_ref[k])
        return dinv * mm(count, dinv * zw) + di2 * zw + b_ref[k:k + 1, :]

    h = jnp.maximum(gcn(x_ref[...], 0), 0.0)
    h = jnp.maximum(gcn(h, 1), 0.0)
    for cell in range(3):
        wi, wc, wo = 2 + 3 * cell, 3 + 3 * cell, 4 + 3 * cell
        gi = jax.nn.sigmoid(mm(h, w_ref[wi]) + b_ref[wi:wi + 1, :])
        gt = jnp.tanh(mm(h, w_ref[wc]) + b_ref[wc:wc + 1, :])
        cn = gi * gt
        wco = b_ref[11 + cell:12 + cell, :]
        go = jax.nn.sigmoid(mm(h, w_ref[wo]) + wco * cn + b_ref[wo:wo + 1, :])
        h = go * jnp.tanh(cn)
    h = jnp.maximum(h, 0.0)
    r0 = mm(h, wj_ref[0]) + bj_ref[0:1, :]
    r1 = mm(h, wj_ref[1]) + bj_ref[1:2, :]
    r2 = mm(h, wj_ref[2]) + bj_ref[2:3, :]
    m = jnp.maximum(r0, jnp.maximum(r1, r2))
    lse = m + jnp.log(jnp.exp(r0 - m) + jnp.exp(r1 - m) + jnp.exp(r2 - m))
    out_ref[0] = r0 - lse
    out_ref[1] = r1 - lse
    out_ref[2] = r2 - lse


def kernel(x, edge_index, no_pedestrians, params):
    src = edge_index[0].astype(jnp.int32)
    dst = edge_index[1].astype(jnp.int32)
    zeros_h = jnp.zeros((_N * _N,), jnp.float32)
    ones_h = jnp.ones((_CHUNK,), jnp.float32)
    counts = _make_sc_count()(src, dst, zeros_h, ones_h).reshape(_NC, _N, _N)

    p = params
    ws = [p['gcn1_W'], p['gcn2_W']]
    bs = [p['gcn1_b'], p['gcn2_b']]
    for g in p['lstms']:
        ws += [g['W_i'], g['W_c'], g['W_o']]
        bs += [g['tb_i'] + g['b_i'][0], g['tb_c'] + g['b_c'][0],
               g['tb_o'] + g['b_o'][0]]
    for g in p['lstms']:
        bs.append(g['wc_o'][0])
    bs += [jnp.zeros((_F,), jnp.float32)] * 2
    w_stack = jnp.stack(ws)
    b_stack = jnp.stack(bs)
    wj = jnp.stack([jnp.pad(p['lin_W'][:, j::3], ((0, 0), (0, 125)))
                    for j in range(3)])
    bj = jnp.stack([jnp.pad(p['lin_b'][j::3], (0, 125))
                    for j in range(3)])
    bj = jnp.pad(bj, ((0, 5), (0, 0)))

    out3 = pl.pallas_call(
        _tc_body,
        out_shape=jax.ShapeDtypeStruct((3, _N, 128), jnp.float32),
    )(counts, x, w_stack, b_stack, wj, bj)

    res = jnp.transpose(out3, (1, 2, 0))[:, :3, :]
    zero = (jnp.asarray(no_pedestrians) - _N).astype(res.dtype)
    return res + zero


# speedup vs baseline: 118.9902x; 1.4401x over previous
import functools

import jax
import jax.numpy as jnp
from jax import lax
from jax.experimental import pallas as pl
from jax.experimental.pallas import tpu as pltpu
from jax.experimental.pallas import tpu_sc as plsc

_N = 512
_F = 512
_E = 131072
_NC = 2
_NS = 16
_NW = _NC * _NS
_EPW = _E // _NW
_CHUNK = 128
_NROW = _EPW // _CHUNK
_SLICE = (_N * _N) // _NS
_ZB = 2048


def _sc_count_body(ei_hbm, out_hbm, src_v, dst_v, idx_v, ones_v, zb_v,
                   shared, sem):
    c = lax.axis_index("c")
    s = lax.axis_index("s")
    wid = s * _NC + c
    base = wid * _EPW
    in_src = pltpu.async_copy(ei_hbm.at[pl.ds(base, _EPW)], src_v, sem)
    in_dst = pltpu.async_copy(ei_hbm.at[pl.ds(_E + base, _EPW)], dst_v, sem)

    ones16 = jnp.full((16,), 1.0, jnp.float32)
    zeros16 = jnp.zeros((16,), jnp.float32)
    for j in range(_CHUNK // 16):
        ones_v[pl.ds(j * 16, 16)] = ones16

    def zbody(i, carry):
        zb_v[pl.ds(i * 16, 16)] = zeros16
        return carry

    lax.fori_loop(0, _ZB // 16, zbody, 0)
    for j in range(_SLICE // _ZB):
        pltpu.sync_copy(zb_v, shared.at[pl.ds(s * _SLICE + j * _ZB, _ZB)])

    in_src.wait()
    in_dst.wait()

    def body(i, carry):
        sv = src_v[pl.ds(i * 16, 16)]
        dv = dst_v[pl.ds(i * 16, 16)]
        row = i // 8
        col = (i % 8) * 16
        idx_v[row, pl.ds(col, 16)] = dv * _N + sv
        return carry

    lax.fori_loop(0, _EPW // 16, body, 0)

    plsc.subcore_barrier()
    copies = [pltpu.async_copy(ones_v, shared.at[idx_v.at[j]], sem, add=True)
              for j in range(_NROW)]
    for cp in copies:
        cp.wait()
    plsc.subcore_barrier()
    pltpu.sync_copy(shared.at[pl.ds(s * _SLICE, _SLICE)],
                    out_hbm.at[c, pl.ds(s * _SLICE, _SLICE)])


@functools.lru_cache(maxsize=1)
def _make_sc_count():
    return functools.partial(
        pl.kernel,
        mesh=plsc.VectorSubcoreMesh(core_axis_name="c", subcore_axis_name="s"),
        out_type=jax.ShapeDtypeStruct((_NC, _N * _N), jnp.float32),
        scratch_types=[
            pltpu.VMEM((_EPW,), jnp.int32),
            pltpu.VMEM((_EPW,), jnp.int32),
            pltpu.VMEM((_NROW, _CHUNK), jnp.int32),
            pltpu.VMEM((_CHUNK,), jnp.float32),
            pltpu.VMEM((_ZB,), jnp.float32),
            pltpu.VMEM_SHARED((_N * _N,), jnp.float32),
            pltpu.SemaphoreType.DMA,
        ],
    )(_sc_count_body)


def _tc_body(counts_ref, x_ref, w1_ref, w2_ref, wg0, wg1, wg2, wg3, wg4,
             wg5, wg6, wg7, wg8, b_ref, wj_ref, bj_ref, out_ref):
    count = counts_ref[0] + counts_ref[1]
    deg = jnp.sum(count, axis=1, keepdims=True) + 2.0
    dinv = lax.rsqrt(deg)
    di2 = 2.0 * dinv * dinv
    gates = (wg0, wg1, wg2, wg3, wg4, wg5, wg6, wg7, wg8)

    def mm(a, b):
        return jnp.dot(a, b, preferred_element_type=jnp.float32)

    def gcn(z, w, k):
        zw = mm(z, w[...])
        return dinv * mm(count, dinv * zw) + di2 * zw + b_ref[k:k + 1, :]

    h = jnp.maximum(gcn(x_ref[...], w1_ref, 0), 0.0)
    h = jnp.maximum(gcn(h, w2_ref, 1), 0.0)
    for cell in range(3):
        wi, wc, wo = 2 + 3 * cell, 3 + 3 * cell, 4 + 3 * cell
        gi = jax.nn.sigmoid(mm(h, gates[3 * cell][...]) + b_ref[wi:wi + 1, :])
        gt = jnp.tanh(mm(h, gates[3 * cell + 1][...]) + b_ref[wc:wc + 1, :])
        cn = gi * gt
        wco = b_ref[11 + cell:12 + cell, :]
        go = jax.nn.sigmoid(mm(h, gates[3 * cell + 2][...]) + wco * cn
                            + b_ref[wo:wo + 1, :])
        h = go * jnp.tanh(cn)
    h = jnp.maximum(h, 0.0)
    r0 = mm(h, wj_ref[0]) + bj_ref[0:1, :]
    r1 = mm(h, wj_ref[1]) + bj_ref[1:2, :]
    r2 = mm(h, wj_ref[2]) + bj_ref[2:3, :]
    m = jnp.maximum(r0, jnp.maximum(r1, r2))
    lse = m + jnp.log(jnp.exp(r0 - m) + jnp.exp(r1 - m) + jnp.exp(r2 - m))
    for j, rj in enumerate((r0, r1, r2)):
        oj = rj - lse
        for g in range(3):
            out_ref[:, 3 * g + j:3 * g + j + 1] = oj[:, g:g + 1]


def kernel(x, edge_index, no_pedestrians, params):
    ei_flat = edge_index.astype(jnp.int32).reshape(-1)
    counts = _make_sc_count()(ei_flat).reshape(_NC, _N, _N)

    p = params
    bs = [p['gcn1_b'], p['gcn2_b']]
    gate_ws = []
    for g in p['lstms']:
        gate_ws += [g['W_i'], g['W_c'], g['W_o']]
        bs += [g['tb_i'] + g['b_i'][0], g['tb_c'] + g['b_c'][0],
               g['tb_o'] + g['b_o'][0]]
    for g in p['lstms']:
        bs.append(g['wc_o'][0])
    bs += [jnp.zeros((_F,), jnp.float32)] * 2
    b_stack = jnp.stack(bs)
    wj = jnp.stack([jnp.pad(p['lin_W'][:, j::3], ((0, 0), (0, 125)))
                    for j in range(3)])
    bj = jnp.stack([jnp.pad(p['lin_b'][j::3], (0, 125))
                    for j in range(3)])
    bj = jnp.pad(bj, ((0, 5), (0, 0)))

    out = pl.pallas_call(
        _tc_body,
        out_shape=jax.ShapeDtypeStruct((_N, 128), jnp.float32),
    )(counts, x, p['gcn1_W'], p['gcn2_W'], *gate_ws, b_stack, wj, bj)

    res = out[:, :9].reshape(_N, 3, 3)
    zero = (jnp.asarray(no_pedestrians) - _N).astype(res.dtype)
    return res + zero
